# baseline (speedup 1.0000x reference)
import jax
import jax.numpy as jnp
from jax.experimental import pallas as pl
from jax.experimental.pallas import tpu as pltpu

_B, _S, _D = 4, 4096, 1024
_R = 1024
_NCH = (_B * _S) // _R
_NBUF = 6


def _body(in_hbm, pos_hbm, out_hbm, *scratch):
    bufs = scratch[:_NBUF]
    pos_v = scratch[_NBUF]
    sp0, sp1 = scratch[_NBUF + 1:_NBUF + 3]
    sis = scratch[_NBUF + 3:2 * _NBUF + 3]
    sos = scratch[2 * _NBUF + 3:3 * _NBUF + 3]

    ins = [pltpu.make_async_copy(
        in_hbm.at[pl.ds(c * _R, _R), :], bufs[c % _NBUF], sis[c % _NBUF])
        for c in range(_NCH)]
    outs = [pltpu.make_async_copy(
        bufs[c % _NBUF], out_hbm.at[pl.ds(c * _R, _R), :], sos[c % _NBUF])
        for c in range(_NCH)]
    _H = _S // 2
    pos_cps = [pltpu.make_async_copy(
        pos_hbm.at[pl.ds(h * _H, _H), :], pos_v.at[pl.ds(h * _H, _H), :], s)
        for h, s in ((0, sp0), (1, sp1))]

    pos_cps[0].start()
    ins[0].start()
    pos_cps[1].start()
    for c in range(1, _NBUF - 1):
        ins[c].start()

    for c in range(_NCH):
        j = c % _NBUF
        if c < len(pos_cps):
            pos_cps[c].wait()
        ins[c].wait()
        p0 = (c % (_S // _R)) * _R
        bufs[j][...] = bufs[j][...] + pos_v[pl.ds(p0, _R), :]
        outs[c].start()
        if c >= 1 and c + _NBUF - 1 < _NCH:
            outs[c - 1].wait()
        if c + _NBUF - 1 < _NCH:
            ins[c + _NBUF - 1].start()

    for c in range(_NCH - _NBUF, _NCH):
        outs[c].wait()


def kernel(inputs, pos_table):
    b, s, d = inputs.shape
    out = pl.pallas_call(
        _body,
        in_specs=[
            pl.BlockSpec(memory_space=pl.ANY),
            pl.BlockSpec(memory_space=pl.ANY),
        ],
        out_specs=pl.BlockSpec(memory_space=pl.ANY),
        out_shape=jax.ShapeDtypeStruct((b * s, d), inputs.dtype),
        scratch_shapes=(
            [pltpu.VMEM((_R, _D), jnp.float32)] * _NBUF
            + [pltpu.VMEM((_S, _D), jnp.float32)]
            + [pltpu.SemaphoreType.DMA] * (2 * _NBUF + 2)
        ),
    )(inputs.reshape(b * s, d), pos_table)
    return out.reshape(b, s, d)


# speedup vs baseline: 1.0089x; 1.0089x over previous
import jax
import jax.numpy as jnp
from jax.experimental import pallas as pl
from jax.experimental.pallas import tpu as pltpu

_B, _S, _D = 4, 4096, 1024
_R = 2048
_NCH = (_B * _S) // _R
_NBUF = 4


def _body(in_hbm, pos_hbm, out_hbm, b0, b1, b2, b3, pos_v,
          sp0, sp1, si0, si1, si2, si3, so0, so1, so2, so3):
    bufs = (b0, b1, b2, b3)
    sis = (si0, si1, si2, si3)
    sos = (so0, so1, so2, so3)

    ins = [pltpu.make_async_copy(
        in_hbm.at[pl.ds(c * _R, _R), :], bufs[c % _NBUF], sis[c % _NBUF])
        for c in range(_NCH)]
    outs = [pltpu.make_async_copy(
        bufs[c % _NBUF], out_hbm.at[pl.ds(c * _R, _R), :], sos[c % _NBUF])
        for c in range(_NCH)]
    pos_cps = [pltpu.make_async_copy(
        pos_hbm.at[pl.ds(h * _R, _R), :], pos_v.at[pl.ds(h * _R, _R), :], s)
        for h, s in ((0, sp0), (1, sp1))]

    pos_cps[0].start()
    ins[0].start()
    pos_cps[1].start()
    for c in range(1, _NBUF - 1):
        ins[c].start()

    for c in range(_NCH):
        j = c % _NBUF
        if c < len(pos_cps):
            pos_cps[c].wait()
        ins[c].wait()
        p0 = (c % (_S // _R)) * _R
        bufs[j][...] = bufs[j][...] + pos_v[pl.ds(p0, _R), :]
        outs[c].start()
        if c >= 1 and c + _NBUF - 1 < _NCH:
            outs[c - 1].wait()
        if c + _NBUF - 1 < _NCH:
            ins[c + _NBUF - 1].start()

    for c in range(_NCH - _NBUF, _NCH):
        outs[c].wait()


def kernel(inputs, pos_table):
    b, s, d = inputs.shape
    out = pl.pallas_call(
        _body,
        in_specs=[
            pl.BlockSpec(memory_space=pl.ANY),
            pl.BlockSpec(memory_space=pl.ANY),
        ],
        out_specs=pl.BlockSpec(memory_space=pl.ANY),
        out_shape=jax.ShapeDtypeStruct((b * s, d), inputs.dtype),
        scratch_shapes=(
            [pltpu.VMEM((_R, _D), jnp.float32)] * _NBUF
            + [pltpu.VMEM((_S, _D), jnp.float32)]
            + [pltpu.SemaphoreType.DMA] * (2 * _NBUF + 2)
        ),
    )(inputs.reshape(b * s, d), pos_table)
    return out.reshape(b, s, d)
